# baseline (speedup 1.0000x reference)
import functools

import jax
import jax.numpy as jnp
from jax import lax
from jax.experimental import pallas as pl
from jax.experimental.pallas import tpu as pltpu
from jax.experimental.pallas import tpu_sc as plsc

VOCAB = 1000000
DIM = 64
N = 16384 * 20

NC = 2
NS = 16
L = 16
NW = NC * NS
BPW = N // NW
C = 512
NBUF = 2
NCH = BPW // C
assert NCH % NBUF == 0

_mesh = plsc.VectorSubcoreMesh(core_axis_name="c", subcore_axis_name="s")


@functools.partial(
    pl.kernel,
    mesh=_mesh,
    compiler_params=pltpu.CompilerParams(use_tc_tiling_on_sc=False),
    out_type=jax.ShapeDtypeStruct((N, DIM), jnp.float32),
    scratch_types=[
        pltpu.VMEM((BPW,), jnp.int32),
        pltpu.VMEM((NBUF, C, DIM), jnp.float32),
        pltpu.SemaphoreType.DMA,
        pltpu.SemaphoreType.DMA,
        pltpu.SemaphoreType.DMA,
        pltpu.SemaphoreType.DMA,
    ],
)
def _emb_lookup(idx_hbm, table_hbm, out_hbm, idx_v, rows_v, g0s, g1s, o0s, o1s):
    gsems = (g0s, g1s)
    osems = (o0s, o1s)
    wid = lax.axis_index("s") * NC + lax.axis_index("c")
    base = wid * BPW

    pltpu.sync_copy(idx_hbm.at[pl.ds(base, BPW)], idx_v)

    vocab = jnp.full((L,), VOCAB, jnp.int32)

    def mod_chunk(g):
        def body(i, carry):
            s = pl.ds(g * C + i * L, L)
            idx_v[s] = lax.rem(idx_v[s], vocab)
            return carry
        lax.fori_loop(0, C // L, body, 0)

    def start_gather(g, b):
        pltpu.async_copy(
            table_hbm.at[idx_v.at[pl.ds(g * C, C)]], rows_v.at[b], gsems[b]
        )

    def wait_gather(g, b):
        pltpu.make_async_copy(
            table_hbm.at[idx_v.at[pl.ds(g * C, C)]], rows_v.at[b], gsems[b]
        ).wait()

    def start_out(g, b):
        pltpu.async_copy(
            rows_v.at[b], out_hbm.at[pl.ds(base + g * C, C)], osems[b]
        )

    def wait_out(g, b):
        pltpu.make_async_copy(
            rows_v.at[b], out_hbm.at[pl.ds(base + g * C, C)], osems[b]
        ).wait()

    for b in range(NBUF):
        mod_chunk(b)
        start_gather(b, b)

    def steady(g0, carry):
        for b in range(NBUF):
            g = g0 * NBUF + b
            wait_gather(g, b)
            start_out(g, b)
            mod_chunk(g + NBUF)
            wait_out(g, b)
            start_gather(g + NBUF, b)
        return carry

    lax.fori_loop(0, (NCH - NBUF) // NBUF, steady, 0)

    for b in range(NBUF):
        g = NCH - NBUF + b
        wait_gather(g, b)
        start_out(g, b)
        wait_out(g, b)


def kernel(indices, table):
    return _emb_lookup(indices.astype(jnp.int32), table)
